# baseline (speedup 1.0000x reference)
import jax
import jax.numpy as jnp
from jax import lax
from jax.experimental import pallas as pl
from jax.experimental.pallas import tpu as pltpu
from jax.experimental.pallas import tpu_sc as plsc

B = 16384
D = 32
H = 64
NROWS = 1000000

_info = plsc.get_sparse_core_info()
_NC = _info.num_cores
_NS = _info.num_subcores
NW = _NC * _NS
BPW = B // NW
IDXW = 128
NCHUNK = BPW // IDXW
NLANE = 16


def _gather_body(uidx_hbm, iidx_hbm, utab_hbm, itab_hbm, uout_hbm, iout_hbm,
                 uidx_v, iidx_v, ufidx_v, ifidx_v, urow_v, irow_v, sem):
    wid = lax.axis_index("s") * _NC + lax.axis_index("c")
    pltpu.sync_copy(uidx_hbm.at[wid], uidx_v)
    pltpu.sync_copy(iidx_hbm.at[wid], iidx_v)

    def build(d, _):
        off = (d * NROWS).astype(jnp.int32)
        for c in range(NCHUNK):
            for k in range(IDXW // NLANE):
                sl = pl.ds(k * NLANE, NLANE)
                ufidx_v[d, c, sl] = uidx_v[c, sl] + off
                ifidx_v[d, c, sl] = iidx_v[c, sl] + off
        return ()

    lax.fori_loop(0, D, build, (), unroll=False)

    def gather(d, _):
        copies = []
        for c in range(NCHUNK):
            copies.append(pltpu.async_copy(
                utab_hbm.at[ufidx_v.at[d, c]], urow_v.at[d, pl.ds(c * IDXW, IDXW)], sem))
            copies.append(pltpu.async_copy(
                itab_hbm.at[ifidx_v.at[d, c]], irow_v.at[d, pl.ds(c * IDXW, IDXW)], sem))
        for cp in copies:
            cp.wait()
        return ()

    lax.fori_loop(0, D, gather, (), unroll=False)

    pltpu.sync_copy(urow_v, uout_hbm.at[wid])
    pltpu.sync_copy(irow_v, iout_hbm.at[wid])


_gather = pl.kernel(
    _gather_body,
    out_type=[
        jax.ShapeDtypeStruct((NW, D, BPW), jnp.float32),
        jax.ShapeDtypeStruct((NW, D, BPW), jnp.float32),
    ],
    mesh=plsc.VectorSubcoreMesh(core_axis_name="c", subcore_axis_name="s"),
    scratch_types=[
        pltpu.VMEM((NCHUNK, IDXW), jnp.int32),
        pltpu.VMEM((NCHUNK, IDXW), jnp.int32),
        pltpu.VMEM((D, NCHUNK, IDXW), jnp.int32),
        pltpu.VMEM((D, NCHUNK, IDXW), jnp.int32),
        pltpu.VMEM((D, BPW), jnp.float32),
        pltpu.VMEM((D, BPW), jnp.float32),
        pltpu.SemaphoreType.DMA,
    ],
    compiler_params=pltpu.CompilerParams(use_tc_tiling_on_sc=False),
)


def _mlp_body(u_ref, i_ref, w1u_ref, w1i_ref, b1_ref, w2_ref, b2_ref, o_ref):
    cdims = (((0,), (0,)), ((), ()))
    h = lax.dot_general(u_ref[0], w1u_ref[...], cdims,
                        preferred_element_type=jnp.float32)
    h = h + lax.dot_general(i_ref[0], w1i_ref[...], cdims,
                            preferred_element_type=jnp.float32)
    h = jnp.maximum(h + b1_ref[...], 0.0)
    o_ref[...] = jnp.dot(h, w2_ref[...],
                         preferred_element_type=jnp.float32) + b2_ref[...]


def kernel(user, item, user_table, item_table, W1, b1, W2, b2):
    uidx = user.astype(jnp.int32).reshape(NW, NCHUNK, IDXW)
    iidx = item.astype(jnp.int32).reshape(NW, NCHUNK, IDXW)
    utab_flat = user_table.T.reshape(D * NROWS)
    itab_flat = item_table.T.reshape(D * NROWS)
    uvec3, ivec3 = _gather(uidx, iidx, utab_flat, itab_flat)

    out = pl.pallas_call(
        _mlp_body,
        grid=(NW,),
        in_specs=[
            pl.BlockSpec((1, D, BPW), lambda g: (g, 0, 0)),
            pl.BlockSpec((1, D, BPW), lambda g: (g, 0, 0)),
            pl.BlockSpec((D, H), lambda g: (0, 0)),
            pl.BlockSpec((D, H), lambda g: (0, 0)),
            pl.BlockSpec((1, H), lambda g: (0, 0)),
            pl.BlockSpec((H, 1), lambda g: (0, 0)),
            pl.BlockSpec((1, 1), lambda g: (0, 0)),
        ],
        out_specs=pl.BlockSpec((BPW, 1), lambda g: (g, 0)),
        out_shape=jax.ShapeDtypeStruct((B, 1), jnp.float32),
    )(uvec3, ivec3, W1[:D], W1[D:], b1.reshape(1, H), W2, b2.reshape(1, 1))
    return out[:, 0]


# speedup vs baseline: 50.3333x; 50.3333x over previous
import jax
import jax.numpy as jnp
from jax.experimental import pallas as pl

B = 16384
D = 32
H = 64


def _mlp_body(u_ref, i_ref, w1u_ref, w1i_ref, b1_ref, w2_ref, b2_ref, o_ref):
    h = jnp.dot(u_ref[...], w1u_ref[...], preferred_element_type=jnp.float32)
    h = h + jnp.dot(i_ref[...], w1i_ref[...], preferred_element_type=jnp.float32)
    h = jnp.maximum(h + b1_ref[...], 0.0)
    o_ref[...] = jnp.dot(h, w2_ref[...],
                         preferred_element_type=jnp.float32) + b2_ref[...]


def kernel(user, item, user_table, item_table, W1, b1, W2, b2):
    uvec = jnp.take(user_table, user, axis=0)
    ivec = jnp.take(item_table, item, axis=0)

    BLK = 2048
    out = pl.pallas_call(
        _mlp_body,
        grid=(B // BLK,),
        in_specs=[
            pl.BlockSpec((BLK, D), lambda g: (g, 0)),
            pl.BlockSpec((BLK, D), lambda g: (g, 0)),
            pl.BlockSpec((D, H), lambda g: (0, 0)),
            pl.BlockSpec((D, H), lambda g: (0, 0)),
            pl.BlockSpec((1, H), lambda g: (0, 0)),
            pl.BlockSpec((H, 1), lambda g: (0, 0)),
            pl.BlockSpec((1, 1), lambda g: (0, 0)),
        ],
        out_specs=pl.BlockSpec((BLK, 1), lambda g: (g, 0)),
        out_shape=jax.ShapeDtypeStruct((B, 1), jnp.float32),
    )(uvec, ivec, W1[:D], W1[D:], b1.reshape(1, H), W2, b2.reshape(1, 1))
    return out[:, 0]
